# baseline (speedup 1.0000x reference)
import functools
import jax
import jax.numpy as jnp
from jax import lax
from jax.experimental import pallas as pl
from jax.experimental.pallas import tpu as pltpu
from jax.experimental.pallas import tpu_sc as plsc

D = 64
CHUNK = 128
SCALE = 8.0
LANES = 16


def _make_emb_kernel(B: int, n_chunks: int, b_per_w: int, num_cores: int):
    mesh = plsc.VectorSubcoreMesh(core_axis_name="c", subcore_axis_name="s")

    @functools.partial(
        pl.kernel,
        out_type=jax.ShapeDtypeStruct((B, D), jnp.float32),
        mesh=mesh,
        scratch_types=[
            pltpu.VMEM((n_chunks, CHUNK), jnp.int32),
            pltpu.VMEM((CHUNK, D), jnp.float32),
            pltpu.SemaphoreType.DMA,
        ],
        compiler_params=pltpu.CompilerParams(use_tc_tiling_on_sc=False),
    )
    def _emb(x_hbm, table_hbm, out_hbm, idx_v, rows_v, sem):
        wid = lax.axis_index("s") * num_cores + lax.axis_index("c")
        base = wid * b_per_w
        pltpu.sync_copy(x_hbm.at[wid], idx_v)

        def chunk_body(j, carry):
            pltpu.async_copy(table_hbm.at[idx_v.at[j]], rows_v, sem).wait()

            def row_body(i, c):
                for q in range(D // LANES):
                    sl = pl.ds(q * LANES, LANES)
                    rows_v[i, sl] = rows_v[i, sl] * SCALE
                return c

            lax.fori_loop(0, CHUNK, row_body, 0)
            pltpu.sync_copy(rows_v, out_hbm.at[pl.ds(base + j * CHUNK, CHUNK)])
            return carry

        lax.fori_loop(0, n_chunks, chunk_body, 0)

    return _emb


@jax.jit
def kernel(x, table):
    info = plsc.get_sparse_core_info()
    nw = info.num_cores * info.num_subcores
    B = x.size
    b_per_w = B // nw
    n_chunks = b_per_w // CHUNK
    x_blocks = x.reshape(nw, n_chunks, CHUNK).astype(jnp.int32)
    emb = _make_emb_kernel(B, n_chunks, b_per_w, info.num_cores)
    out = emb(x_blocks, table)
    return out.reshape(*x.shape, D)


# speedup vs baseline: 1.1828x; 1.1828x over previous
import functools
import jax
import jax.numpy as jnp
from jax import lax
from jax.experimental import pallas as pl
from jax.experimental.pallas import tpu as pltpu
from jax.experimental.pallas import tpu_sc as plsc

D = 64
CHUNK = 128
SCALE = 8.0
LANES = 16
NBUF = 4
LEAD = 2


def _make_emb_kernel(B: int, n_chunks: int, b_per_w: int, num_cores: int):
    assert n_chunks % NBUF == 0 and n_chunks >= 2 * NBUF
    n_groups = n_chunks // NBUF
    mesh = plsc.VectorSubcoreMesh(core_axis_name="c", subcore_axis_name="s")

    @functools.partial(
        pl.kernel,
        out_type=jax.ShapeDtypeStruct((B, D), jnp.float32),
        mesh=mesh,
        scratch_types=[
            pltpu.VMEM((n_chunks, CHUNK), jnp.int32),
            pltpu.VMEM((NBUF, CHUNK, D), jnp.float32),
            pltpu.SemaphoreType.DMA((NBUF,)),
            pltpu.SemaphoreType.DMA((NBUF,)),
        ],
        compiler_params=pltpu.CompilerParams(use_tc_tiling_on_sc=False),
    )
    def _emb(x_hbm, table_hbm, out_hbm, idx_v, rows, gsem, osem):
        wid = lax.axis_index("s") * num_cores + lax.axis_index("c")
        base = wid * b_per_w
        pltpu.sync_copy(x_hbm.at[wid], idx_v)

        def start_gather(j, b):
            pltpu.async_copy(table_hbm.at[idx_v.at[j]], rows.at[b], gsem.at[b])

        def wait_gather(j, b):
            pltpu.make_async_copy(
                table_hbm.at[idx_v.at[j]], rows.at[b], gsem.at[b]
            ).wait()

        def start_out(j, b):
            pltpu.async_copy(
                rows.at[b], out_hbm.at[pl.ds(base + j * CHUNK, CHUNK)], osem.at[b]
            )

        def wait_out(j, b):
            pltpu.make_async_copy(
                rows.at[b], out_hbm.at[pl.ds(base + j * CHUNK, CHUNK)], osem.at[b]
            ).wait()

        def scale(b):
            def row_body(i, c):
                for q in range(D // LANES):
                    sl = pl.ds(q * LANES, LANES)
                    rows[b, i, sl] = rows[b, i, sl] * SCALE
                return c

            lax.fori_loop(0, CHUNK, row_body, 0, unroll=2)

        def process(j, b):
            wait_gather(j, b)
            scale(b)
            start_out(j, b)

        for j in range(LEAD):
            start_gather(j, j % NBUF)

        for b in range(NBUF):
            process(b, b)
            nj = b + LEAD
            if b >= LEAD:
                wait_out(nj - NBUF, nj % NBUF)
            start_gather(nj, nj % NBUF)

        def group_body(g, carry):
            j0 = g * NBUF
            for b in range(NBUF):
                j = j0 + b
                process(j, b)
                nb = (b + LEAD) % NBUF
                wait_out(j + LEAD - NBUF, nb)
                start_gather(j + LEAD, nb)
            return carry

        lax.fori_loop(1, n_groups - 1, group_body, 0)

        j0 = (n_groups - 1) * NBUF
        for b in range(NBUF):
            j = j0 + b
            process(j, b)
            nj = j + LEAD
            if nj < n_chunks:
                wait_out(nj - NBUF, nj % NBUF)
                start_gather(nj, nj % NBUF)

        for b in range(NBUF):
            wait_out(j0 + b, b)

    return _emb


@jax.jit
def kernel(x, table):
    info = plsc.get_sparse_core_info()
    nw = info.num_cores * info.num_subcores
    B = x.size
    b_per_w = B // nw
    n_chunks = b_per_w // CHUNK
    x_blocks = x.reshape(nw, n_chunks, CHUNK).astype(jnp.int32)
    emb = _make_emb_kernel(B, n_chunks, b_per_w, info.num_cores)
    out = emb(x_blocks, table)
    return out.reshape(*x.shape, D)
